# baseline (speedup 1.0000x reference)
import jax
import jax.numpy as jnp
from jax import lax
from jax.experimental import pallas as pl
from jax.experimental.pallas import tpu as pltpu
from jax.experimental.pallas import tpu_sc as plsc

_TOKENS = 8192
_DIM = 2048
_NE = 16
_EPS = 1e-06
_CAP = float(_TOKENS)
_MM_TILE = 2048
_MM_GRID = _TOKENS // _MM_TILE
_NW = 16
_TPW = _TOKENS // _NW


def _mm_body(x_ref, w_ref, b_ref, out_ref):
    out_ref[...] = lax.dot_general(
        x_ref[...], w_ref[...], (((1,), (1,)), ((), ())),
        preferred_element_type=jnp.float32,
    ) + b_ref[...]


def _route_body(logits_hbm, out_hbm, lv, mv, av, sv):
    wid = lax.axis_index("s")
    base = wid * _TPW
    pltpu.sync_copy(logits_hbm.at[pl.ds(base, _TPW)], lv)

    idx = lax.iota(jnp.int32, 16)

    def one(t):
        v = lv[t, :]
        m = jnp.max(v)
        e = jnp.exp(v - m)
        s = jnp.sum(e)
        p = e / s
        pm = jnp.full((16,), 1.0, jnp.float32) / s
        first = plsc.all_reduce_ffs(p == pm)
        masked = jnp.where(idx == first, p, 0.0)
        mv[t, :] = masked
        return masked

    def body(i, acc):
        t = i * 8
        a = [one(t + u) for u in range(8)]
        s01 = (a[0] + a[1]) + (a[2] + a[3])
        s23 = (a[4] + a[5]) + (a[6] + a[7])
        return acc + (s01 + s23)

    acc = lax.fori_loop(0, _TPW // 8, body, jnp.zeros((16,), jnp.float32))
    av[:] = acc
    pltpu.sync_copy(av, out_hbm.at[wid])
    plsc.subcore_barrier()
    pltpu.sync_copy(out_hbm.at[pl.ds(0, _NW)], sv)
    plsc.subcore_barrier()
    tot = sv[0, :]
    for j in range(1, _NW):
        tot = tot + sv[j, :]
    scale = _CAP / (tot + _EPS)

    def body2(i, carry):
        t = i * 8
        for u in range(8):
            mv[t + u, :] = mv[t + u, :] * scale
        return carry

    lax.fori_loop(0, _TPW // 8, body2, 0)
    pltpu.sync_copy(mv, out_hbm.at[pl.ds(base, _TPW)])


def kernel(x, W, b):
    b2 = b.reshape(1, _NE)
    logits = pl.pallas_call(
        _mm_body,
        grid=(_MM_GRID,),
        in_specs=[
            pl.BlockSpec((_MM_TILE, _DIM), lambda i: (i, 0)),
            pl.BlockSpec((_NE, _DIM), lambda i: (0, 0)),
            pl.BlockSpec((1, _NE), lambda i: (0, 0)),
        ],
        out_specs=pl.BlockSpec((_MM_TILE, _NE), lambda i: (i, 0)),
        out_shape=jax.ShapeDtypeStruct((_TOKENS, _NE), jnp.float32),
    )(x, W, b2)

    route = pl.kernel(
        _route_body,
        mesh=plsc.VectorSubcoreMesh(
            core_axis_name="c", subcore_axis_name="s", num_cores=1
        ),
        compiler_params=pltpu.CompilerParams(needs_layout_passes=False, use_tc_tiling_on_sc=False),
        out_type=jax.ShapeDtypeStruct((_TOKENS, _NE), jnp.float32),
        scratch_types=[
            pltpu.VMEM((_TPW, _NE), jnp.float32),
            pltpu.VMEM((_TPW, _NE), jnp.float32),
            pltpu.VMEM((_NE,), jnp.float32),
            pltpu.VMEM((_NW, _NE), jnp.float32),
        ],
    )
    return route(logits)


# speedup vs baseline: 1.0347x; 1.0347x over previous
import jax
import jax.numpy as jnp
from jax import lax
from jax.experimental import pallas as pl
from jax.experimental.pallas import tpu as pltpu
from jax.experimental.pallas import tpu_sc as plsc

_TOKENS = 8192
_DIM = 2048
_NE = 16
_EPS = 1e-06
_CAP = float(_TOKENS)
_MM_TILE = 1024
_MM_GRID = _TOKENS // _MM_TILE
_NW = 16
_TPW = _TOKENS // _NW


def _mm_body(x_ref, w_ref, b_ref, out_ref):
    out_ref[...] = lax.dot_general(
        x_ref[...], w_ref[...], (((1,), (1,)), ((), ())),
        preferred_element_type=jnp.float32,
    ) + b_ref[...]


def _route_body(logits_hbm, out_hbm, lv, mv, av, sv):
    wid = lax.axis_index("s")
    base = wid * _TPW
    pltpu.sync_copy(logits_hbm.at[pl.ds(base, _TPW)], lv)

    idx = lax.iota(jnp.int32, 16)

    def one(t):
        v = lv[t, :]
        m = jnp.max(v)
        e = jnp.exp(v - m)
        s = jnp.sum(e)
        p = e / s
        pm = jnp.full((16,), 1.0, jnp.float32) / s
        first = plsc.all_reduce_ffs(p == pm)
        masked = jnp.where(idx == first, p, 0.0)
        mv[t, :] = masked
        return masked

    def body(i, acc):
        t = i * 8
        a = [one(t + u) for u in range(8)]
        s01 = (a[0] + a[1]) + (a[2] + a[3])
        s23 = (a[4] + a[5]) + (a[6] + a[7])
        return acc + (s01 + s23)

    acc = lax.fori_loop(0, _TPW // 8, body, jnp.zeros((16,), jnp.float32))
    av[:] = acc
    pltpu.sync_copy(av, out_hbm.at[wid])
    plsc.subcore_barrier()
    pltpu.sync_copy(out_hbm.at[pl.ds(0, _NW)], sv)
    plsc.subcore_barrier()
    tot = sv[0, :]
    for j in range(1, _NW):
        tot = tot + sv[j, :]
    scale = _CAP / (tot + _EPS)

    def body2(i, carry):
        t = i * 8
        for u in range(8):
            mv[t + u, :] = mv[t + u, :] * scale
        return carry

    lax.fori_loop(0, _TPW // 8, body2, 0)
    pltpu.sync_copy(mv, out_hbm.at[pl.ds(base, _TPW)])


def kernel(x, W, b):
    b2 = b.reshape(1, _NE)
    logits = pl.pallas_call(
        _mm_body,
        grid=(_MM_GRID,),
        in_specs=[
            pl.BlockSpec((_MM_TILE, _DIM), lambda i: (i, 0)),
            pl.BlockSpec((_NE, _DIM), lambda i: (0, 0)),
            pl.BlockSpec((1, _NE), lambda i: (0, 0)),
        ],
        out_specs=pl.BlockSpec((_MM_TILE, _NE), lambda i: (i, 0)),
        out_shape=jax.ShapeDtypeStruct((_TOKENS, _NE), jnp.float32),
    )(x, W, b2)

    route = pl.kernel(
        _route_body,
        mesh=plsc.VectorSubcoreMesh(
            core_axis_name="c", subcore_axis_name="s", num_cores=1
        ),
        compiler_params=pltpu.CompilerParams(needs_layout_passes=False, use_tc_tiling_on_sc=False),
        out_type=jax.ShapeDtypeStruct((_TOKENS, _NE), jnp.float32),
        scratch_types=[
            pltpu.VMEM((_TPW, _NE), jnp.float32),
            pltpu.VMEM((_TPW, _NE), jnp.float32),
            pltpu.VMEM((_NE,), jnp.float32),
            pltpu.VMEM((_NW, _NE), jnp.float32),
        ],
    )
    return route(logits)
